# baseline (speedup 1.0000x reference)
import functools

import jax
import jax.numpy as jnp
from jax import lax
from jax.experimental import pallas as pl
from jax.experimental.pallas import tpu as pltpu
from jax.experimental.pallas import tpu_sc as plsc

B, S, D, N = 4, 8192, 1024, 50
NR = N - 1
WPB = 8
NW = B * WPB
SLOTS = 8
CHUNKS = D // 16
PAD = 64


def _sc_gather_dots(table_hbm, idx_hbm, out_hbm, idx_v, rows_v, res_v, sem):
    wid = lax.axis_index("s") * 2 + lax.axis_index("c")
    pltpu.sync_copy(idx_hbm.at[wid], idx_v)
    pltpu.async_copy(table_hbm.at[idx_v], rows_v, sem).wait()

    zero = jnp.zeros((16,), jnp.float32)

    def chunk_body(c, carry):
        nums, rns = carry
        dchunk = rows_v[SLOTS - 1, pl.ds(c * 16, 16)]
        new_nums = []
        new_rns = []
        for j in range(SLOTS - 1):
            rchunk = rows_v[j, pl.ds(c * 16, 16)]
            new_nums.append(nums[j] + rchunk * dchunk)
            new_rns.append(rns[j] + rchunk * rchunk)
        new_nums.append(nums[SLOTS - 1] + dchunk * dchunk)
        return tuple(new_nums), tuple(new_rns)

    nums0 = tuple(zero for _ in range(SLOTS))
    rns0 = tuple(zero for _ in range(SLOTS - 1))
    nums, rns = lax.fori_loop(0, CHUNKS, chunk_body, (nums0, rns0))

    lane = lax.iota(jnp.int32, 16)

    dnums = lax.GatherDimensionNumbers(
        offset_dims=(), collapsed_slice_dims=(0,), start_index_map=(0,))

    def lane_shuffle(x, perm):
        return lax.gather(
            x, perm[:, None], dnums, slice_sizes=(1,),
            mode=lax.GatherScatterMode.PROMISE_IN_BOUNDS,
            unique_indices=True)

    def lane_sum(x):
        for s in (1, 2, 4, 8):
            x = x + lane_shuffle(x, lane ^ s)
        return x

    res = jnp.zeros((16,), jnp.float32)
    for j in range(SLOTS):
        res = jnp.where(lane == j, lane_sum(nums[j]), res)
    for j in range(SLOTS - 1):
        res = jnp.where(lane == (SLOTS + j), lane_sum(rns[j]), res)
    res_v[...] = res
    pltpu.sync_copy(res_v, out_hbm.at[wid])


_sc_call = functools.partial(
    pl.kernel,
    mesh=plsc.VectorSubcoreMesh(core_axis_name="c", subcore_axis_name="s"),
    out_type=jax.ShapeDtypeStruct((NW, 16), jnp.float32),
    scratch_types=[
        pltpu.VMEM((SLOTS,), jnp.int32),
        pltpu.VMEM((SLOTS, D), jnp.float32),
        pltpu.VMEM((16,), jnp.float32),
        pltpu.SemaphoreType.DMA,
    ],
)(_sc_gather_dots)


def _tc_finish(num_ref, rn2_ref, dn2_ref, lab_ref, sim_ref, loss_ref):
    num = num_ref[...]
    rn2 = rn2_ref[...]
    dn2 = dn2_ref[...]
    col = lax.broadcasted_iota(jnp.int32, (B, PAD), 1)
    valid = col < NR
    denom = jnp.maximum(jnp.sqrt(rn2 * dn2), 1e-6)
    sim = jnp.where(valid, num / denom, 0.0)
    sim_ref[...] = sim
    logits = jnp.where(valid, sim, -1e30)
    m = jnp.max(logits, axis=1, keepdims=True)
    lse = jnp.log(jnp.sum(jnp.exp(logits - m), axis=1, keepdims=True)) + m
    picked = jnp.sum(jnp.where(col == lab_ref[...], sim, 0.0),
                     axis=1, keepdims=True)
    loss_ref[...] = jnp.sum(lse - picked, axis=0, keepdims=True) / B


def kernel(sequence_output, sep_positions, labels):
    table = sequence_output.reshape(B * S, D)
    flat = sep_positions.astype(jnp.int32) + (
        jnp.arange(B, dtype=jnp.int32) * S)[:, None]
    u = jnp.arange(WPB)[:, None]
    k = jnp.arange(SLOTS - 1)[None, :]
    j = u + WPB * k
    jc = jnp.where(j < NR, j, NR)
    jfull = jnp.concatenate(
        [jc, jnp.full((WPB, 1), NR, jnp.int32)], axis=1)
    idx = flat[:, jfull].reshape(NW, SLOTS)

    out = _sc_call(table, idx)
    out = out.reshape(B, WPB, 16)
    dn2 = out[:, 0, SLOTS - 1][:, None]
    numm = out[:, :, 0:SLOTS - 1].transpose(0, 2, 1).reshape(B, 56)[:, :NR]
    rn2m = out[:, :, SLOTS:2 * SLOTS - 1].transpose(0, 2, 1).reshape(B, 56)[:, :NR]
    pad = ((0, 0), (0, PAD - NR))
    numm = jnp.pad(numm, pad)
    rn2m = jnp.pad(rn2m, pad, constant_values=1.0)

    sim_full, loss = pl.pallas_call(
        _tc_finish,
        out_shape=[
            jax.ShapeDtypeStruct((B, PAD), jnp.float32),
            jax.ShapeDtypeStruct((1, 1), jnp.float32),
        ],
    )(numm, rn2m, dn2, labels.astype(jnp.int32)[:, None])
    return sim_full[:, :NR], loss[0, 0]


# speedup vs baseline: 1.0813x; 1.0813x over previous
import functools

import jax
import jax.numpy as jnp
from jax import lax
from jax.experimental import pallas as pl
from jax.experimental.pallas import tpu as pltpu
from jax.experimental.pallas import tpu_sc as plsc

B, S, D, N = 4, 8192, 1024, 50
NR = N - 1
WPB = 8
NW = B * WPB
RPW = 7
SLOTS = 8
CHUNKS = D // 16


def _sc_gather_dots(table_hbm, sep_hbm, out_hbm, sep_v, idx_v, rows_v,
                    res_v, sem):
    wid = lax.axis_index("s") * 2 + lax.axis_index("c")
    b = wid // WPB
    u = wid % WPB

    pltpu.sync_copy(sep_hbm, sep_v.at[pl.ds(0, B * N)])
    lane = lax.iota(jnp.int32, 16)

    dnums = lax.GatherDimensionNumbers(
        offset_dims=(), collapsed_slice_dims=(0,), start_index_map=(0,))

    def lane_shuffle(x, perm):
        return lax.gather(
            x, perm[:, None], dnums, slice_sizes=(1,),
            mode=lax.GatherScatterMode.PROMISE_IN_BOUNDS,
            unique_indices=True)

    win = sep_v[pl.ds(b * N + u * RPW, 16)]
    win_d = sep_v[pl.ds(b * N + NR, 16)]
    dsel = lane_shuffle(win_d, jnp.zeros((16,), jnp.int32))
    limit = jnp.where(u < RPW, RPW, 0)
    vals = jnp.where(lane < limit, win, dsel)
    idx_v[...] = jnp.clip(vals + b * S, 0, B * S - 1)

    pltpu.async_copy(table_hbm.at[idx_v.at[pl.ds(0, SLOTS)]], rows_v,
                     sem).wait()

    zero = jnp.zeros((16,), jnp.float32)

    def chunk_body(c, carry):
        nums, rns = carry
        dchunk = rows_v[SLOTS - 1, pl.ds(c * 16, 16)]
        new_nums = []
        new_rns = []
        for r in range(SLOTS - 1):
            rchunk = rows_v[r, pl.ds(c * 16, 16)]
            new_nums.append(nums[r] + rchunk * dchunk)
            new_rns.append(rns[r] + rchunk * rchunk)
        new_nums.append(nums[SLOTS - 1] + dchunk * dchunk)
        return tuple(new_nums), tuple(new_rns)

    nums0 = tuple(zero for _ in range(SLOTS))
    rns0 = tuple(zero for _ in range(SLOTS - 1))
    nums, rns = lax.fori_loop(0, CHUNKS, chunk_body, (nums0, rns0))

    def lane_sum(x):
        for s in (1, 2, 4, 8):
            x = x + lane_shuffle(x, lane ^ s)
        return x

    res = jnp.zeros((16,), jnp.float32)
    for r in range(SLOTS):
        res = jnp.where(lane == r, lane_sum(nums[r]), res)
    for r in range(SLOTS - 1):
        res = jnp.where(lane == (SLOTS + r), lane_sum(rns[r]), res)
    res_v[...] = res
    pltpu.sync_copy(res_v, out_hbm.at[wid])


_sc_call = functools.partial(
    pl.kernel,
    mesh=plsc.VectorSubcoreMesh(core_axis_name="c", subcore_axis_name="s"),
    out_type=jax.ShapeDtypeStruct((NW, 16), jnp.float32),
    scratch_types=[
        pltpu.VMEM((B * N + 24,), jnp.int32),
        pltpu.VMEM((16,), jnp.int32),
        pltpu.VMEM((SLOTS, D), jnp.float32),
        pltpu.VMEM((16,), jnp.float32),
        pltpu.SemaphoreType.DMA,
    ],
)(_sc_gather_dots)


def _tc_finish(o_ref, lab_ref, sim_ref, loss_ref):
    o = o_ref[...]
    num = o[:, 0:SLOTS]
    rn2 = o[:, SLOTS:2 * SLOTS]
    dn2 = o[:, SLOTS - 1:SLOTS]
    r_i = lax.broadcasted_iota(jnp.int32, (NW, SLOTS), 0)
    k_i = lax.broadcasted_iota(jnp.int32, (NW, SLOTS), 1)
    u_i = lax.rem(r_i, WPB)
    valid = (u_i < RPW) & (k_i < RPW)
    raw = num / jnp.maximum(jnp.sqrt(rn2 * dn2), 1e-6)
    sim = jnp.where(valid, raw, 0.0)
    sim_ref[...] = sim
    e = jnp.where(valid, jnp.exp(sim), 0.0)
    esum_row = jnp.sum(e, axis=1, keepdims=True)
    a_r = lax.broadcasted_iota(jnp.int32, (B, NW), 0)
    a_c = lax.broadcasted_iota(jnp.int32, (B, NW), 1)
    agg = (lax.div(a_c, WPB) == a_r).astype(jnp.float32)
    t_r = lax.broadcasted_iota(jnp.int32, (NW, B), 0)
    t_c = lax.broadcasted_iota(jnp.int32, (NW, B), 1)
    spread = (lax.div(t_r, WPB) == t_c).astype(jnp.float32)
    esum_b = jax.lax.dot(agg, esum_row,
                         preferred_element_type=jnp.float32)
    lse = jnp.log(esum_b)
    labrow = jax.lax.dot(spread, lab_ref[...].astype(jnp.float32),
                         preferred_element_type=jnp.float32)
    jpos = (u_i * RPW + k_i).astype(jnp.float32)
    match = valid & (jpos == labrow)
    picked_row = jnp.sum(jnp.where(match, sim, 0.0), axis=1,
                         keepdims=True)
    picked_b = jax.lax.dot(agg, picked_row,
                           preferred_element_type=jnp.float32)
    loss_ref[...] = jnp.sum(lse - picked_b, axis=0, keepdims=True) / B


def kernel(sequence_output, sep_positions, labels):
    table = sequence_output.reshape(B * S, D)
    sep_flat = sep_positions.astype(jnp.int32).reshape(B * N)
    out = _sc_call(table, sep_flat)
    sim8, loss = pl.pallas_call(
        _tc_finish,
        out_shape=[
            jax.ShapeDtypeStruct((NW, SLOTS), jnp.float32),
            jax.ShapeDtypeStruct((1, 1), jnp.float32),
        ],
    )(out, labels.astype(jnp.int32)[:, None])
    sim_scores = sim8.reshape(B, WPB, SLOTS)[:, :RPW, :RPW].reshape(B, NR)
    return sim_scores, loss[0, 0]


# speedup vs baseline: 1.0851x; 1.0035x over previous
import functools

import jax
import jax.numpy as jnp
from jax import lax
from jax.experimental import pallas as pl
from jax.experimental.pallas import tpu as pltpu
from jax.experimental.pallas import tpu_sc as plsc

B, S, D, N = 4, 8192, 1024, 50
NR = N - 1
WPB = 8
NW = B * WPB
RPW = 7
SLOTS = 8
CHUNKS = D // 16


def _sc_gather_dots(table_hbm, sep_hbm, out_hbm, sep_v, idx_v, rows_v,
                    res_v, sem):
    wid = lax.axis_index("s") * 2 + lax.axis_index("c")
    b = wid // WPB
    u = wid % WPB

    pltpu.sync_copy(sep_hbm, sep_v)
    lane = lax.iota(jnp.int32, 16)

    dnums = lax.GatherDimensionNumbers(
        offset_dims=(), collapsed_slice_dims=(0,), start_index_map=(0,))

    def lane_shuffle(x, perm):
        return lax.gather(
            x, perm[:, None], dnums, slice_sizes=(1,),
            mode=lax.GatherScatterMode.PROMISE_IN_BOUNDS,
            unique_indices=True)

    win = sep_v[b, pl.ds(u * RPW, 16)]
    win_d = sep_v[b, pl.ds(NR + 0 * u, 16)]
    dsel = lane_shuffle(win_d, jnp.zeros((16,), jnp.int32))
    limit = jnp.where(u < RPW, RPW, 0)
    vals = jnp.where(lane < limit, win, dsel)
    idx_v[...] = jnp.clip(vals + b * S, 0, B * S - 1)

    pltpu.async_copy(table_hbm.at[idx_v.at[pl.ds(0, SLOTS)]], rows_v,
                     sem).wait()

    zero = jnp.zeros((16,), jnp.float32)

    def acc_chunk(c, nums, rns):
        dchunk = rows_v[SLOTS - 1, pl.ds(c * 16, 16)]
        new_nums = []
        new_rns = []
        for r in range(SLOTS - 1):
            rchunk = rows_v[r, pl.ds(c * 16, 16)]
            new_nums.append(nums[r] + rchunk * dchunk)
            new_rns.append(rns[r] + rchunk * rchunk)
        new_nums.append(nums[SLOTS - 1] + dchunk * dchunk)
        return new_nums, new_rns

    def chunk_body(c, carry):
        nums, rns = carry
        nums, rns = acc_chunk(2 * c, nums, rns)
        nums, rns = acc_chunk(2 * c + 1, nums, rns)
        return tuple(nums), tuple(rns)

    nums0 = tuple(zero for _ in range(SLOTS))
    rns0 = tuple(zero for _ in range(SLOTS - 1))
    nums, rns = lax.fori_loop(0, CHUNKS // 2, chunk_body, (nums0, rns0))

    vecs = list(nums) + list(rns) + [zero]
    s = 1
    while len(vecs) > 1:
        nxt = []
        for i in range(0, len(vecs), 2):
            a, b2 = vecs[i], vecs[i + 1]
            ra = a + lane_shuffle(a, lane ^ s)
            rb = b2 + lane_shuffle(b2, lane ^ s)
            nxt.append(jnp.where((lane & s) == 0, ra, rb))
        vecs = nxt
        s *= 2
    res_v[...] = vecs[0]
    pltpu.sync_copy(res_v, out_hbm.at[wid])


_sc_call = functools.partial(
    pl.kernel,
    mesh=plsc.VectorSubcoreMesh(core_axis_name="c", subcore_axis_name="s"),
    out_type=jax.ShapeDtypeStruct((NW, 16), jnp.float32),
    scratch_types=[
        pltpu.VMEM((B, N), jnp.int32),
        pltpu.VMEM((16,), jnp.int32),
        pltpu.VMEM((SLOTS, D), jnp.float32),
        pltpu.VMEM((16,), jnp.float32),
        pltpu.SemaphoreType.DMA,
    ],
)(_sc_gather_dots)


def _tc_finish(o_ref, lab_ref, sim_ref, loss_ref):
    o = o_ref[...]
    num = o[:, 0:SLOTS]
    rn2 = o[:, SLOTS:2 * SLOTS]
    dn2 = o[:, SLOTS - 1:SLOTS]
    r_i = lax.broadcasted_iota(jnp.int32, (NW, SLOTS), 0)
    k_i = lax.broadcasted_iota(jnp.int32, (NW, SLOTS), 1)
    u_i = lax.rem(r_i, WPB)
    valid = (u_i < RPW) & (k_i < RPW)
    raw = num / jnp.maximum(jnp.sqrt(rn2 * dn2), 1e-6)
    sim = jnp.where(valid, raw, 0.0)
    sim_ref[...] = sim
    e = jnp.where(valid, jnp.exp(sim), 0.0)
    esum_row = jnp.sum(e, axis=1, keepdims=True)
    a_r = lax.broadcasted_iota(jnp.int32, (B, NW), 0)
    a_c = lax.broadcasted_iota(jnp.int32, (B, NW), 1)
    agg = (lax.div(a_c, WPB) == a_r).astype(jnp.float32)
    t_r = lax.broadcasted_iota(jnp.int32, (NW, B), 0)
    t_c = lax.broadcasted_iota(jnp.int32, (NW, B), 1)
    spread = (lax.div(t_r, WPB) == t_c).astype(jnp.float32)
    esum_b = jax.lax.dot(agg, esum_row,
                         preferred_element_type=jnp.float32)
    lse = jnp.log(esum_b)
    labrow = jax.lax.dot(spread, lab_ref[...].astype(jnp.float32),
                         preferred_element_type=jnp.float32)
    jpos = (u_i * RPW + k_i).astype(jnp.float32)
    match = valid & (jpos == labrow)
    picked_row = jnp.sum(jnp.where(match, sim, 0.0), axis=1,
                         keepdims=True)
    picked_b = jax.lax.dot(agg, picked_row,
                           preferred_element_type=jnp.float32)
    loss_ref[...] = jnp.sum(lse - picked_b, axis=0, keepdims=True) / B


def kernel(sequence_output, sep_positions, labels):
    table = sequence_output.reshape(B * S, D)
    out = _sc_call(table, sep_positions.astype(jnp.int32))
    sim8, loss = pl.pallas_call(
        _tc_finish,
        out_shape=[
            jax.ShapeDtypeStruct((NW, SLOTS), jnp.float32),
            jax.ShapeDtypeStruct((1, 1), jnp.float32),
        ],
    )(out, labels.astype(jnp.int32)[:, None])
    sim_scores = sim8.reshape(B, WPB, SLOTS)[:, :RPW, :RPW].reshape(B, NR)
    return sim_scores, loss[0, 0]


# speedup vs baseline: 1.1480x; 1.0580x over previous
import functools

import jax
import jax.numpy as jnp
from jax import lax
from jax.experimental import pallas as pl
from jax.experimental.pallas import tpu as pltpu
from jax.experimental.pallas import tpu_sc as plsc

B, S, D, N = 4, 8192, 1024, 50
NR = N - 1
WPB = 8
NW = B * WPB
RPW = 7
SLOTS = 8
CHUNKS = D // 16


def _sc_gather_dots(table_hbm, sep_hbm, out_hbm, sep_v, idx_v, rows_v,
                    res_v, sem):
    wid = lax.axis_index("s") * 2 + lax.axis_index("c")
    b = wid // WPB
    u = wid % WPB

    pltpu.sync_copy(sep_hbm, sep_v)
    lane = lax.iota(jnp.int32, 16)

    dnums = lax.GatherDimensionNumbers(
        offset_dims=(), collapsed_slice_dims=(0,), start_index_map=(0,))

    def lane_shuffle(x, perm):
        return lax.gather(
            x, perm[:, None], dnums, slice_sizes=(1,),
            mode=lax.GatherScatterMode.PROMISE_IN_BOUNDS,
            unique_indices=True)

    win = sep_v[b, pl.ds(u * RPW, 16)]
    win_d = sep_v[b, pl.ds(NR + 0 * u, 16)]
    dsel = lane_shuffle(win_d, jnp.zeros((16,), jnp.int32))
    limit = jnp.where(u < RPW, RPW, 0)
    vals = jnp.where(lane < limit, win, dsel)
    idx_v[...] = jnp.clip(vals + b * S, 0, B * S - 1)

    pltpu.async_copy(table_hbm.at[idx_v.at[pl.ds(0, SLOTS)]], rows_v,
                     sem).wait()

    zero = jnp.zeros((16,), jnp.float32)

    def acc_chunk(c, nums, rns):
        dchunk = rows_v[SLOTS - 1, pl.ds(c * 16, 16)]
        new_nums = []
        new_rns = []
        for r in range(SLOTS - 1):
            rchunk = rows_v[r, pl.ds(c * 16, 16)]
            new_nums.append(nums[r] + rchunk * dchunk)
            new_rns.append(rns[r] + rchunk * rchunk)
        new_nums.append(nums[SLOTS - 1] + dchunk * dchunk)
        return new_nums, new_rns

    def chunk_body(c, carry):
        nums, rns = carry
        nums, rns = acc_chunk(2 * c, nums, rns)
        nums, rns = acc_chunk(2 * c + 1, nums, rns)
        return tuple(nums), tuple(rns)

    nums0 = tuple(zero for _ in range(SLOTS))
    rns0 = tuple(zero for _ in range(SLOTS - 1))
    nums, rns = lax.fori_loop(0, CHUNKS // 2, chunk_body, (nums0, rns0))

    vecs = list(nums) + list(rns) + [zero]
    s = 1
    while len(vecs) > 1:
        nxt = []
        for i in range(0, len(vecs), 2):
            a, b2 = vecs[i], vecs[i + 1]
            ra = a + lane_shuffle(a, lane ^ s)
            rb = b2 + lane_shuffle(b2, lane ^ s)
            nxt.append(jnp.where((lane & s) == 0, ra, rb))
        vecs = nxt
        s *= 2
    res_v[...] = vecs[0]
    pltpu.sync_copy(res_v.at[pl.ds(0, 8)],
                    out_hbm.at[b, pl.ds(8 * u, 8)])
    pltpu.sync_copy(res_v.at[pl.ds(8, 8)],
                    out_hbm.at[b, pl.ds(64 + 8 * u, 8)])


_sc_call = functools.partial(
    pl.kernel,
    mesh=plsc.VectorSubcoreMesh(core_axis_name="c", subcore_axis_name="s"),
    out_type=jax.ShapeDtypeStruct((B, 128), jnp.float32),
    scratch_types=[
        pltpu.VMEM((B, N), jnp.int32),
        pltpu.VMEM((16,), jnp.int32),
        pltpu.VMEM((SLOTS, D), jnp.float32),
        pltpu.VMEM((16,), jnp.float32),
        pltpu.SemaphoreType.DMA,
    ],
)(_sc_gather_dots)


def _tc_finish(o_ref, lab_ref, sim_ref, loss_ref):
    o = o_ref[...]
    num = o[:, 0:64]
    rn2 = o[:, 64:128]
    dn2 = num[:, RPW:SLOTS]
    col = lax.broadcasted_iota(jnp.int32, (B, 64), 1)
    uu = lax.div(col, WPB)
    kk = lax.rem(col, WPB)
    valid = (uu < RPW) & (kk < RPW)
    raw = num / jnp.maximum(jnp.sqrt(rn2 * dn2), 1e-6)
    sim = jnp.where(valid, raw, 0.0)
    e = jnp.where(valid, jnp.exp(sim), 0.0)
    lse = jnp.log(jnp.sum(e, axis=1, keepdims=True))
    jpos = uu * RPW + kk
    match = valid & (jpos == lab_ref[...])
    picked = jnp.sum(jnp.where(match, sim, 0.0), axis=1,
                     keepdims=True)
    loss_ref[...] = jnp.sum(lse - picked, axis=0, keepdims=True) / B
    p_c = lax.broadcasted_iota(jnp.int32, (64, NR), 0)
    p_j = lax.broadcasted_iota(jnp.int32, (64, NR), 1)
    perm = (p_c == WPB * lax.div(p_j, RPW)
            + lax.rem(p_j, RPW)).astype(jnp.float32)
    sim_ref[...] = jax.lax.dot(sim, perm,
                               preferred_element_type=jnp.float32)


def kernel(sequence_output, sep_positions, labels):
    table = sequence_output.reshape(B * S, D)
    out = _sc_call(table, sep_positions.astype(jnp.int32))
    sim_scores, loss = pl.pallas_call(
        _tc_finish,
        out_shape=[
            jax.ShapeDtypeStruct((B, NR), jnp.float32),
            jax.ShapeDtypeStruct((1, 1), jnp.float32),
        ],
    )(out, labels.astype(jnp.int32)[:, None])
    return sim_scores, loss[0, 0]


# speedup vs baseline: 1.1576x; 1.0084x over previous
import functools

import jax
import jax.numpy as jnp
from jax import lax
from jax.experimental import pallas as pl
from jax.experimental.pallas import tpu as pltpu
from jax.experimental.pallas import tpu_sc as plsc

B, S, D, N = 4, 8192, 1024, 50
NR = N - 1
WPB = 8
NW = B * WPB
RPW = 7
SLOTS = 8
CHUNKS = D // 16


def _sc_gather_dots(table_hbm, sep_hbm, out_hbm, sep_v, idx_v, rows_v,
                    res_v, sem):
    wid = lax.axis_index("s") * 2 + lax.axis_index("c")
    b = wid // WPB
    u = wid % WPB

    pltpu.sync_copy(sep_hbm, sep_v)
    lane = lax.iota(jnp.int32, 16)

    dnums = lax.GatherDimensionNumbers(
        offset_dims=(), collapsed_slice_dims=(0,), start_index_map=(0,))

    def lane_shuffle(x, perm):
        return lax.gather(
            x, perm[:, None], dnums, slice_sizes=(1,),
            mode=lax.GatherScatterMode.PROMISE_IN_BOUNDS,
            unique_indices=True)

    win = sep_v[b, pl.ds(u * RPW, 16)]
    win_d = sep_v[b, pl.ds(NR + 0 * u, 16)]
    dsel = lane_shuffle(win_d, jnp.zeros((16,), jnp.int32))
    limit = jnp.where(u < RPW, RPW, 0)
    vals = jnp.where(lane < limit, win, dsel)
    idx_v[...] = jnp.clip(vals + b * S, 0, B * S - 1)

    pltpu.async_copy(table_hbm.at[idx_v.at[pl.ds(0, SLOTS)]], rows_v,
                     sem).wait()

    zero = jnp.zeros((16,), jnp.float32)

    def acc_chunk(c, nums, rns):
        dchunk = rows_v[SLOTS - 1, pl.ds(c * 16, 16)]
        new_nums = []
        new_rns = []
        for r in range(SLOTS - 1):
            rchunk = rows_v[r, pl.ds(c * 16, 16)]
            new_nums.append(nums[r] + rchunk * dchunk)
            new_rns.append(rns[r] + rchunk * rchunk)
        new_nums.append(nums[SLOTS - 1] + dchunk * dchunk)
        return new_nums, new_rns

    def chunk_body(c, carry):
        nums, rns = carry
        nums, rns = acc_chunk(2 * c, nums, rns)
        nums, rns = acc_chunk(2 * c + 1, nums, rns)
        return tuple(nums), tuple(rns)

    nums0 = tuple(zero for _ in range(SLOTS))
    rns0 = tuple(zero for _ in range(SLOTS - 1))
    nums, rns = lax.fori_loop(0, CHUNKS // 2, chunk_body, (nums0, rns0))

    vecs = list(nums) + list(rns) + [zero]
    s = 1
    while len(vecs) > 1:
        nxt = []
        for i in range(0, len(vecs), 2):
            a, b2 = vecs[i], vecs[i + 1]
            ra = a + lane_shuffle(a, lane ^ s)
            rb = b2 + lane_shuffle(b2, lane ^ s)
            nxt.append(jnp.where((lane & s) == 0, ra, rb))
        vecs = nxt
        s *= 2
    res_v[...] = vecs[0]
    pltpu.sync_copy(res_v.at[pl.ds(0, 8)],
                    out_hbm.at[b, pl.ds(8 * u, 8)])
    pltpu.sync_copy(res_v.at[pl.ds(8, 8)],
                    out_hbm.at[b, pl.ds(64 + 8 * u, 8)])


_sc_call = functools.partial(
    pl.kernel,
    mesh=plsc.VectorSubcoreMesh(core_axis_name="c", subcore_axis_name="s"),
    out_type=jax.ShapeDtypeStruct((B, 128), jnp.float32),
    scratch_types=[
        pltpu.VMEM((B, N), jnp.int32),
        pltpu.VMEM((16,), jnp.int32),
        pltpu.VMEM((SLOTS, D), jnp.float32),
        pltpu.VMEM((16,), jnp.float32),
        pltpu.SemaphoreType.DMA,
    ],
)(_sc_gather_dots)


def _tc_finish(o_ref, lab_ref, sim_ref, loss_ref):
    o = o_ref[...]
    num = o[:, 0:64]
    rn2 = o[:, 64:128]
    dn2 = num[:, RPW:SLOTS]
    col = lax.broadcasted_iota(jnp.int32, (B, 64), 1)
    uu = lax.div(col, WPB)
    kk = lax.rem(col, WPB)
    valid = (uu < RPW) & (kk < RPW)
    raw = num / jnp.maximum(jnp.sqrt(rn2 * dn2), 1e-6)
    sim = jnp.where(valid, raw, 0.0)
    e = jnp.where(valid, jnp.exp(sim), 0.0)
    lse = jnp.log(jnp.sum(e, axis=1, keepdims=True))
    jpos = uu * RPW + kk
    match = valid & (jpos == lab_ref[...])
    picked = jnp.sum(jnp.where(match, sim, 0.0), axis=1,
                     keepdims=True)
    loss_ref[...] = jnp.sum(lse - picked, axis=0, keepdims=True) / B
    p_c = lax.broadcasted_iota(jnp.int32, (64, NR), 0)
    p_j = lax.broadcasted_iota(jnp.int32, (64, NR), 1)
    perm = (p_c == WPB * lax.div(p_j, RPW)
            + lax.rem(p_j, RPW)).astype(jnp.float32)
    sim_ref[...] = jax.lax.dot(sim, perm,
                               precision=jax.lax.Precision.HIGHEST,
                               preferred_element_type=jnp.float32)


def kernel(sequence_output, sep_positions, labels):
    table = sequence_output.reshape(B * S, D)
    out = _sc_call(table, sep_positions.astype(jnp.int32))
    sim_scores, loss = pl.pallas_call(
        _tc_finish,
        out_shape=[
            jax.ShapeDtypeStruct((B, NR), jnp.float32),
            jax.ShapeDtypeStruct((1, 1), jnp.float32),
        ],
    )(out, labels.astype(jnp.int32)[:, None])
    return sim_scores, loss[0, 0]
